# baseline (speedup 1.0000x reference)
import functools

import jax
import jax.numpy as jnp
from jax import lax
from jax.experimental import pallas as pl
from jax.experimental.pallas import tpu as pltpu
from jax.experimental.pallas import tpu_sc as plsc

_N_CFG = 6000
_N_AST = 3800
_N_TEST = 200
_N = _N_CFG + _N_AST + _N_TEST
_E = 320000
_D = 128
_H = _D // 2
_VC = 1000
_VA = 1000
_NO = 10
_NA = 3

_NC = 2
_NS = 16
_NW = _NC * _NS

_N_ACC = 10240
_ECHUNK = 128
_NCHT = 2560
_FAST_CORE = 1
_CF = 128
_CS = 32
_CWIN = 32
_RPT = _N_ACC // _NS
_GPW = _N_ACC // _NW
_GCH = 80
_GN = _GPW // _GCH

@functools.cache
def _sc_kernels():
    mesh = plsc.VectorSubcoreMesh(core_axis_name="c", subcore_axis_name="s")

    nbuf = 2

    @functools.partial(
        pl.kernel,
        mesh=mesh,
        out_type=jax.ShapeDtypeStruct((_NC, _N_ACC, _D), jnp.float32),
        scratch_types=[
            pltpu.VMEM((_CWIN, _ECHUNK), jnp.int32),
            pltpu.VMEM((_CWIN, _ECHUNK), jnp.int32),
            pltpu.VMEM((nbuf, _ECHUNK, _D), jnp.float32),
            pltpu.VMEM((16, _D), jnp.float32),
            pltpu.VMEM_SHARED((_N_ACC, _D), jnp.float32),
        ] + [pltpu.SemaphoreType.DMA] * (2 * nbuf),
    )
    def segsum_k(p_hbm, src_hbm, dst_hbm, out_hbm, src_v, dst_v, gbuf, zbuf, acc, *sems):
        c = lax.axis_index("c")
        s = lax.axis_index("s")
        wid = s * _NC + c
        osems = sems[nbuf:]
        osem = osems[0]

        z16 = jnp.zeros((16,), jnp.float32)

        def _zf(i, carry):
            zbuf[i // 8, pl.ds((i % 8) * 16, 16)] = z16
            return carry

        lax.fori_loop(0, 16 * (_D // 16), _zf, 0)
        nz = _RPT // 16

        def _zc(k, carry):
            pltpu.async_copy(zbuf, acc.at[pl.ds(s * _RPT + k * 16, 16)], osem)
            return carry

        lax.fori_loop(0, nz, _zc, 0)

        def _zw(k, carry):
            pltpu.make_async_copy(zbuf, acc.at[pl.ds(s * _RPT, 16)], osem).wait()
            return carry

        lax.fori_loop(0, nz, _zw, 0)
        plsc.subcore_barrier()

        nst = jnp.where(c == _FAST_CORE, _CF // _CWIN, _CS // _CWIN)

        def _stage(h, carry):
            pltpu.sync_copy(src_hbm.at[wid, pl.ds(h * _CWIN, _CWIN)], src_v)
            pltpu.sync_copy(dst_hbm.at[wid, pl.ds(h * _CWIN, _CWIN)], dst_v)
            for b in range(nbuf):
                pltpu.async_copy(p_hbm.at[src_v.at[b]], gbuf.at[b], sems[b])

            def _step(jj, carry2):
                for b in range(nbuf):
                    j = jj * nbuf + b
                    pltpu.make_async_copy(p_hbm.at[src_v.at[j]], gbuf.at[b], sems[b]).wait()
                    pltpu.sync_copy(gbuf.at[b], acc.at[dst_v.at[j]], add=True)
                    pltpu.async_copy(p_hbm.at[src_v.at[j + nbuf]], gbuf.at[b], sems[b])
                return carry2

            lax.fori_loop(0, _CWIN // nbuf - 1, _step, 0)
            for b in range(nbuf):
                j = _CWIN - nbuf + b
                pltpu.make_async_copy(p_hbm.at[src_v.at[j]], gbuf.at[b], sems[b]).wait()
                pltpu.sync_copy(gbuf.at[b], acc.at[dst_v.at[j]], add=True)
            return carry

        lax.fori_loop(0, nst, _stage, 0)
        plsc.subcore_barrier()

        nout = _RPT // _ECHUNK
        for k in range(nout):
            b = k % nbuf
            r0 = s * _RPT + k * _ECHUNK
            if k >= nbuf:
                pltpu.make_async_copy(
                    gbuf.at[b], out_hbm.at[c, pl.ds(r0, _ECHUNK)], osems[b]).wait()
            pltpu.async_copy(acc.at[pl.ds(r0, _ECHUNK)], gbuf.at[b], sems[b]).wait()
            pltpu.async_copy(gbuf.at[b], out_hbm.at[c, pl.ds(r0, _ECHUNK)], osems[b])
        for k in range(nout - nbuf, nout):
            b = k % nbuf
            r0 = s * _RPT + k * _ECHUNK
            pltpu.make_async_copy(
                gbuf.at[b], out_hbm.at[c, pl.ds(r0, _ECHUNK)], osems[b]).wait()

    @functools.partial(
        pl.kernel,
        mesh=mesh,
        out_type=jax.ShapeDtypeStruct((_N_ACC, _D), jnp.float32),
        scratch_types=[
            pltpu.VMEM((_GN, _GCH), jnp.int32),
            pltpu.VMEM((_GCH, _D), jnp.float32),
            pltpu.SemaphoreType.DMA,
        ],
    )
    def emb_gather_k(tab_hbm, idx_hbm, out_hbm, idx_v, gbuf, sem):
        c = lax.axis_index("c")
        s = lax.axis_index("s")
        wid = s * _NC + c
        pltpu.sync_copy(idx_hbm.at[wid], idx_v)
        for k in range(_GN):
            pltpu.async_copy(tab_hbm.at[idx_v.at[k]], gbuf, sem).wait()
            pltpu.sync_copy(gbuf, out_hbm.at[pl.ds(wid * _GPW + k * _GCH, _GCH)])

    return segsum_k, emb_gather_k


def _segsum(p, src_r, dst_r):
    return _sc_kernels()[0](p, src_r, dst_r)


def _emb_gather(tab, gidx):
    return _sc_kernels()[1](tab, gidx)


_BR = 1024
_NBR = _N_ACC // _BR


def _fuse(a, b, bias, w, res=None, relu=True):
    has_res = res is not None

    def body(*refs):
        if has_res:
            a_r, b_r, bias_r, w_r, res_r, h_r, p_r = refs
        else:
            a_r, b_r, bias_r, w_r, h_r, p_r = refs
        x = a_r[...] + b_r[...] + bias_r[...]
        if relu:
            x = jnp.maximum(x, 0.0)
        if has_res:
            x = x + res_r[...]
        h_r[...] = x
        p_r[...] = jnp.dot(x, w_r[...], preferred_element_type=jnp.float32)

    row = pl.BlockSpec((_BR, _D), lambda i: (i, 0))
    one = pl.BlockSpec((1, _D), lambda i: (0, 0))
    ww = pl.BlockSpec((_D, _D), lambda i: (0, 0))
    in_specs = [row, row, one, ww] + ([row] if has_res else [])
    args = (a, b, bias, w) + ((res,) if has_res else ())
    return pl.pallas_call(
        body,
        grid=(_NBR,),
        in_specs=in_specs,
        out_specs=[row, row],
        out_shape=[jax.ShapeDtypeStruct((_N_ACC, _D), jnp.float32)] * 2,
    )(*args)


def _mm_bias(x, w, b, br):
    rows = x.shape[0]

    def body(x_r, w_r, b_r, o_r):
        o_r[...] = jnp.dot(x_r[...], w_r[...], preferred_element_type=jnp.float32) + b_r[...]

    return pl.pallas_call(
        body,
        grid=(rows // br,),
        in_specs=[
            pl.BlockSpec((br, _D), lambda i: (i, 0)),
            pl.BlockSpec((_D, _D), lambda i: (0, 0)),
            pl.BlockSpec((1, _D), lambda i: (0, 0)),
        ],
        out_specs=pl.BlockSpec((br, _D), lambda i: (i, 0)),
        out_shape=jax.ShapeDtypeStruct((rows, _D), jnp.float32),
    )(x, w, b)


def _head(x, w, b, nvalid):
    def body(x_r, w_r, b_r, l_r, p_r):
        l = jnp.dot(x_r[...], w_r[...], preferred_element_type=jnp.float32) + b_r[...]
        col = lax.broadcasted_iota(jnp.int32, l.shape, 1)
        mask = col < nvalid
        ml = jnp.where(mask, l, -1e30)
        mx = jnp.max(ml, axis=1, keepdims=True)
        e = jnp.where(mask, jnp.exp(ml - mx), 0.0)
        ssum = jnp.sum(e, axis=1, keepdims=True)
        l_r[...] = l
        p_r[...] = e / ssum

    row = pl.BlockSpec((_BR, _D), lambda i: (i, 0))
    return pl.pallas_call(
        body,
        grid=(_NBR,),
        in_specs=[row, pl.BlockSpec((_D, _D), lambda i: (0, 0)),
                  pl.BlockSpec((1, _D), lambda i: (0, 0))],
        out_specs=[row, row],
        out_shape=[jax.ShapeDtypeStruct((_N_ACC, _D), jnp.float32)] * 2,
    )(x, w, b)


def _layout_edges(x, padval):
    nreal = _E // _ECHUNK
    ch = x.reshape(nreal, _ECHUNK)
    padc = jnp.full((_NCHT - nreal, _ECHUNK), padval, jnp.int32)
    nreal_fast = _NS * _CF - (_NCHT - nreal)
    fast = jnp.concatenate([ch[:nreal_fast], padc]).reshape(_NS, _CF, _ECHUNK)
    slow = ch[nreal_fast:].reshape(_NS, _CS, _ECHUNK)
    slow = jnp.concatenate(
        [slow, jnp.zeros((_NS, _CF - _CS, _ECHUNK), jnp.int32)], axis=1)
    parts = [None, None]
    parts[_FAST_CORE] = fast
    parts[1 - _FAST_CORE] = slow
    return jnp.stack(parts, axis=1).reshape(_NW, _CF, _ECHUNK)


def kernel(cfg_label, cfg_content, ast_label, ast_content, edge_index,
           c_lbl_emb, Wc, bc, a_lbl_emb, Wa, ba, t_emb,
           W1, b1, W2, b2, W3, b3, W4, b4, W5, b5,
           Wd, bd, Wad, bad):
    f32 = jnp.float32

    src = edge_index[0].astype(jnp.int32)
    dst = edge_index[1].astype(jnp.int32)
    src_r = _layout_edges(src, 0)
    dst_r = _layout_edges(dst, _N)

    tab = jnp.zeros((2048, _D), f32)
    tab = tab.at[:_VC, :_H].set(c_lbl_emb)
    tab = tab.at[_VC:_VC + _VA, :_H].set(a_lbl_emb)
    tab = tab.at[_VC + _VA, :].set(t_emb)
    gidx = jnp.concatenate([
        cfg_label.astype(jnp.int32),
        ast_label.astype(jnp.int32) + _VC,
        jnp.full((_N_TEST,), _VC + _VA, jnp.int32),
        jnp.full((_N_ACC - _N,), _VC + _VA + 1, jnp.int32),
    ]).reshape(_NW, _GN, _GCH)

    Wc_p = jnp.zeros((_D, _D), f32).at[:, _H:].set(Wc)
    bc_p = jnp.zeros((1, _D), f32).at[0, _H:].set(bc)
    Wa_p = jnp.zeros((_D, _D), f32).at[:, _H:].set(Wa)
    ba_p = jnp.zeros((1, _D), f32).at[0, _H:].set(ba)
    cfgc = jnp.concatenate([cfg_content, jnp.zeros((6144 - _N_CFG, _D), f32)])
    astc = jnp.concatenate([ast_content, jnp.zeros((3840 - _N_AST, _D), f32)])

    g = _emb_gather(tab, gidx)
    cp_cfg = _mm_bias(cfgc, Wc_p, bc_p, 768)
    cp_ast = _mm_bias(astc, Wa_p, ba_p, 768)
    cp = jnp.concatenate([cp_cfg[:_N_CFG], cp_ast[:_N_AST],
                          jnp.zeros((_N_ACC - _N, _D), f32)])

    z128 = jnp.zeros((1, _D), f32)
    _, p1 = _fuse(g, cp, z128, W1, relu=False)
    a1 = _segsum(p1, src_r, dst_r)
    h1, p2 = _fuse(a1[0], a1[1], b1.reshape(1, -1), W2)
    a2 = _segsum(p2, src_r, dst_r)
    h2, p3 = _fuse(a2[0], a2[1], b2.reshape(1, -1), W3, res=h1)
    a3 = _segsum(p3, src_r, dst_r)
    h3, p4 = _fuse(a3[0], a3[1], b3.reshape(1, -1), W4)
    a4 = _segsum(p4, src_r, dst_r)
    h4, p5 = _fuse(a4[0], a4[1], b4.reshape(1, -1), W5, res=h3)
    a5 = _segsum(p5, src_r, dst_r)
    h5, _ = _fuse(a5[0], a5[1], b5.reshape(1, -1), W5)

    Wd_p = jnp.zeros((_D, _D), f32).at[:, :_NO].set(Wd)
    bd_p = jnp.zeros((1, _D), f32).at[0, :_NO].set(bd)
    Wad_p = jnp.zeros((_D, _D), f32).at[:, :_NA].set(Wad)
    bad_p = jnp.zeros((1, _D), f32).at[0, :_NA].set(bad)
    lc, pc = _head(h5, Wd_p, bd_p, _NO)
    la, pa = _head(h5, Wad_p, bad_p, _NA)

    cfg_logits = lc[:_N_CFG, :_NO]
    cfg_pred = pc[:_N_CFG, :_NO]
    ast_logits = la[_N_CFG:_N_CFG + _N_AST, :_NA]
    ast_pred = pa[_N_CFG:_N_CFG + _N_AST, :_NA]
    return (cfg_logits, cfg_pred, ast_logits, ast_pred)


# speedup vs baseline: 1.1487x; 1.1487x over previous
import functools

import jax
import jax.numpy as jnp
from jax import lax
from jax.experimental import pallas as pl
from jax.experimental.pallas import tpu as pltpu
from jax.experimental.pallas import tpu_sc as plsc

_N_CFG = 6000
_N_AST = 3800
_N_TEST = 200
_N = _N_CFG + _N_AST + _N_TEST
_E = 320000
_D = 128
_H = _D // 2
_VC = 1000
_VA = 1000
_NO = 10
_NA = 3

_NC = 2
_NS = 16
_NW = _NC * _NS

_N_ACC = 10240
_ECHUNK = 128
_NCHT = 2560
_FAST_CORE = 1
_CF = 80
_CS = 80
_CWIN = 40
_RPT = _N_ACC // _NS
_GPW = _N_ACC // _NW
_GCH = 80
_GN = _GPW // _GCH

@functools.cache
def _sc_kernels():
    mesh = plsc.VectorSubcoreMesh(core_axis_name="c", subcore_axis_name="s")

    nbuf = 2

    @functools.partial(
        pl.kernel,
        mesh=mesh,
        out_type=jax.ShapeDtypeStruct((_NC, _N_ACC, _D), jnp.float32),
        scratch_types=[
            pltpu.VMEM((_CWIN, _ECHUNK), jnp.int32),
            pltpu.VMEM((_CWIN, _ECHUNK), jnp.int32),
            pltpu.VMEM((nbuf, _ECHUNK, _D), jnp.float32),
            pltpu.VMEM((16, _D), jnp.float32),
            pltpu.VMEM_SHARED((_N_ACC, _D), jnp.float32),
        ] + [pltpu.SemaphoreType.DMA] * (2 * nbuf),
    )
    def segsum_k(p_hbm, src_hbm, dst_hbm, out_hbm, src_v, dst_v, gbuf, zbuf, acc, *sems):
        c = lax.axis_index("c")
        s = lax.axis_index("s")
        wid = s * _NC + c
        osems = sems[nbuf:]
        osem = osems[0]

        z16 = jnp.zeros((16,), jnp.float32)

        def _zf(i, carry):
            zbuf[i // 8, pl.ds((i % 8) * 16, 16)] = z16
            return carry

        lax.fori_loop(0, 16 * (_D // 16), _zf, 0)
        nz = _RPT // 16

        def _zc(k, carry):
            pltpu.async_copy(zbuf, acc.at[pl.ds(s * _RPT + k * 16, 16)], osem)
            return carry

        lax.fori_loop(0, nz, _zc, 0)

        def _zw(k, carry):
            pltpu.make_async_copy(zbuf, acc.at[pl.ds(s * _RPT, 16)], osem).wait()
            return carry

        lax.fori_loop(0, nz, _zw, 0)
        plsc.subcore_barrier()

        nst = jnp.where(c == _FAST_CORE, _CF // _CWIN, _CS // _CWIN)

        def _stage(h, carry):
            pltpu.sync_copy(src_hbm.at[wid, pl.ds(h * _CWIN, _CWIN)], src_v)
            pltpu.sync_copy(dst_hbm.at[wid, pl.ds(h * _CWIN, _CWIN)], dst_v)
            for b in range(nbuf):
                pltpu.async_copy(p_hbm.at[src_v.at[b]], gbuf.at[b], sems[b])

            def _step(jj, carry2):
                for b in range(nbuf):
                    j = jj * nbuf + b
                    pltpu.make_async_copy(p_hbm.at[src_v.at[j]], gbuf.at[b], sems[b]).wait()
                    pltpu.sync_copy(gbuf.at[b], acc.at[dst_v.at[j]], add=True)
                    pltpu.async_copy(p_hbm.at[src_v.at[j + nbuf]], gbuf.at[b], sems[b])
                return carry2

            lax.fori_loop(0, _CWIN // nbuf - 1, _step, 0)
            for b in range(nbuf):
                j = _CWIN - nbuf + b
                pltpu.make_async_copy(p_hbm.at[src_v.at[j]], gbuf.at[b], sems[b]).wait()
                pltpu.sync_copy(gbuf.at[b], acc.at[dst_v.at[j]], add=True)
            return carry

        lax.fori_loop(0, nst, _stage, 0)
        plsc.subcore_barrier()

        nout = _RPT // _ECHUNK
        for k in range(nout):
            b = k % nbuf
            r0 = s * _RPT + k * _ECHUNK
            if k >= nbuf:
                pltpu.make_async_copy(
                    gbuf.at[b], out_hbm.at[c, pl.ds(r0, _ECHUNK)], osems[b]).wait()
            pltpu.async_copy(acc.at[pl.ds(r0, _ECHUNK)], gbuf.at[b], sems[b]).wait()
            pltpu.async_copy(gbuf.at[b], out_hbm.at[c, pl.ds(r0, _ECHUNK)], osems[b])
        for k in range(nout - nbuf, nout):
            b = k % nbuf
            r0 = s * _RPT + k * _ECHUNK
            pltpu.make_async_copy(
                gbuf.at[b], out_hbm.at[c, pl.ds(r0, _ECHUNK)], osems[b]).wait()

    @functools.partial(
        pl.kernel,
        mesh=mesh,
        out_type=jax.ShapeDtypeStruct((_N_ACC, _D), jnp.float32),
        scratch_types=[
            pltpu.VMEM((_GN, _GCH), jnp.int32),
            pltpu.VMEM((_GCH, _D), jnp.float32),
            pltpu.SemaphoreType.DMA,
        ],
    )
    def emb_gather_k(tab_hbm, idx_hbm, out_hbm, idx_v, gbuf, sem):
        c = lax.axis_index("c")
        s = lax.axis_index("s")
        wid = s * _NC + c
        pltpu.sync_copy(idx_hbm.at[wid], idx_v)
        for k in range(_GN):
            pltpu.async_copy(tab_hbm.at[idx_v.at[k]], gbuf, sem).wait()
            pltpu.sync_copy(gbuf, out_hbm.at[pl.ds(wid * _GPW + k * _GCH, _GCH)])

    return segsum_k, emb_gather_k


def _segsum(p, src_r, dst_r):
    return _sc_kernels()[0](p, src_r, dst_r)


def _emb_gather(tab, gidx):
    return _sc_kernels()[1](tab, gidx)


_BR = 1024
_NBR = _N_ACC // _BR


def _fuse(a, b, bias, w, res=None):
    has_res = res is not None

    def body(*refs):
        if has_res:
            a_r, b_r, bias_r, w_r, res_r, h_r = refs
        else:
            a_r, b_r, bias_r, w_r, h_r = refs
        agg = a_r[...] + b_r[...]
        x = jnp.dot(agg, w_r[...], preferred_element_type=jnp.float32) + bias_r[...]
        x = jnp.maximum(x, 0.0)
        if has_res:
            x = x + res_r[...]
        h_r[...] = x

    row = pl.BlockSpec((_BR, _D), lambda i: (i, 0))
    one = pl.BlockSpec((1, _D), lambda i: (0, 0))
    ww = pl.BlockSpec((_D, _D), lambda i: (0, 0))
    in_specs = [row, row, one, ww] + ([row] if has_res else [])
    args = (a, b, bias, w) + ((res,) if has_res else ())
    return pl.pallas_call(
        body,
        grid=(_NBR,),
        in_specs=in_specs,
        out_specs=row,
        out_shape=jax.ShapeDtypeStruct((_N_ACC, _D), jnp.float32),
    )(*args)


def _add2(a, b):
    def body(a_r, b_r, o_r):
        o_r[...] = a_r[...] + b_r[...]

    row = pl.BlockSpec((_BR, _D), lambda i: (i, 0))
    return pl.pallas_call(
        body,
        grid=(_NBR,),
        in_specs=[row, row],
        out_specs=row,
        out_shape=jax.ShapeDtypeStruct((_N_ACC, _D), jnp.float32),
    )(a, b)


def _mm_bias(x, w, b, br):
    rows = x.shape[0]

    def body(x_r, w_r, b_r, o_r):
        o_r[...] = jnp.dot(x_r[...], w_r[...], preferred_element_type=jnp.float32) + b_r[...]

    return pl.pallas_call(
        body,
        grid=(rows // br,),
        in_specs=[
            pl.BlockSpec((br, _D), lambda i: (i, 0)),
            pl.BlockSpec((_D, _D), lambda i: (0, 0)),
            pl.BlockSpec((1, _D), lambda i: (0, 0)),
        ],
        out_specs=pl.BlockSpec((br, _D), lambda i: (i, 0)),
        out_shape=jax.ShapeDtypeStruct((rows, _D), jnp.float32),
    )(x, w, b)


def _head(x, w, b, nvalid):
    def body(x_r, w_r, b_r, l_r, p_r):
        l = jnp.dot(x_r[...], w_r[...], preferred_element_type=jnp.float32) + b_r[...]
        col = lax.broadcasted_iota(jnp.int32, l.shape, 1)
        mask = col < nvalid
        ml = jnp.where(mask, l, -1e30)
        mx = jnp.max(ml, axis=1, keepdims=True)
        e = jnp.where(mask, jnp.exp(ml - mx), 0.0)
        ssum = jnp.sum(e, axis=1, keepdims=True)
        l_r[...] = l
        p_r[...] = e / ssum

    row = pl.BlockSpec((_BR, _D), lambda i: (i, 0))
    return pl.pallas_call(
        body,
        grid=(_NBR,),
        in_specs=[row, pl.BlockSpec((_D, _D), lambda i: (0, 0)),
                  pl.BlockSpec((1, _D), lambda i: (0, 0))],
        out_specs=[row, row],
        out_shape=[jax.ShapeDtypeStruct((_N_ACC, _D), jnp.float32)] * 2,
    )(x, w, b)


def _layout_edges(x, padval):
    nreal = _E // _ECHUNK
    ch = x.reshape(nreal, _ECHUNK)
    padc = jnp.full((_NCHT - nreal, _ECHUNK), padval, jnp.int32)
    nreal_fast = _NS * _CF - (_NCHT - nreal)
    fast = jnp.concatenate([ch[:nreal_fast], padc]).reshape(_NS, _CF, _ECHUNK)
    slow = ch[nreal_fast:].reshape(_NS, _CS, _ECHUNK)
    slow = jnp.concatenate(
        [slow, jnp.full((_NS, _CF - _CS, _ECHUNK), padval, jnp.int32)], axis=1)
    parts = [None, None]
    parts[_FAST_CORE] = fast
    parts[1 - _FAST_CORE] = slow
    return jnp.stack(parts, axis=1).reshape(_NW, _CF, _ECHUNK)


def kernel(cfg_label, cfg_content, ast_label, ast_content, edge_index,
           c_lbl_emb, Wc, bc, a_lbl_emb, Wa, ba, t_emb,
           W1, b1, W2, b2, W3, b3, W4, b4, W5, b5,
           Wd, bd, Wad, bad):
    f32 = jnp.float32

    src = edge_index[0].astype(jnp.int32)
    dst = edge_index[1].astype(jnp.int32)
    src_r = _layout_edges(src, 0)
    dst_r = _layout_edges(dst, _N)

    tab = jnp.zeros((2048, _D), f32)
    tab = tab.at[:_VC, :_H].set(c_lbl_emb)
    tab = tab.at[_VC:_VC + _VA, :_H].set(a_lbl_emb)
    tab = tab.at[_VC + _VA, :].set(t_emb)
    gidx = jnp.concatenate([
        cfg_label.astype(jnp.int32),
        ast_label.astype(jnp.int32) + _VC,
        jnp.full((_N_TEST,), _VC + _VA, jnp.int32),
        jnp.full((_N_ACC - _N,), _VC + _VA + 1, jnp.int32),
    ]).reshape(_NW, _GN, _GCH)

    Wc_p = jnp.zeros((_D, _D), f32).at[:, _H:].set(Wc)
    bc_p = jnp.zeros((1, _D), f32).at[0, _H:].set(bc)
    Wa_p = jnp.zeros((_D, _D), f32).at[:, _H:].set(Wa)
    ba_p = jnp.zeros((1, _D), f32).at[0, _H:].set(ba)
    cfgc = jnp.concatenate([cfg_content, jnp.zeros((6144 - _N_CFG, _D), f32)])
    astc = jnp.concatenate([ast_content, jnp.zeros((3840 - _N_AST, _D), f32)])

    g = _emb_gather(tab, gidx)
    cp_cfg = _mm_bias(cfgc, Wc_p, bc_p, 768)
    cp_ast = _mm_bias(astc, Wa_p, ba_p, 768)
    cp = jnp.concatenate([cp_cfg[:_N_CFG], cp_ast[:_N_AST],
                          jnp.zeros((_N_ACC - _N, _D), f32)])

    h0 = _add2(g, cp)
    a1 = _segsum(h0, src_r, dst_r)
    h1 = _fuse(a1[0], a1[1], b1.reshape(1, -1), W1)
    a2 = _segsum(h1, src_r, dst_r)
    h2 = _fuse(a2[0], a2[1], b2.reshape(1, -1), W2, res=h1)
    a3 = _segsum(h2, src_r, dst_r)
    h3 = _fuse(a3[0], a3[1], b3.reshape(1, -1), W3)
    a4 = _segsum(h3, src_r, dst_r)
    h4 = _fuse(a4[0], a4[1], b4.reshape(1, -1), W4, res=h3)
    a5 = _segsum(h4, src_r, dst_r)
    h5 = _fuse(a5[0], a5[1], b5.reshape(1, -1), W5)

    Wd_p = jnp.zeros((_D, _D), f32).at[:, :_NO].set(Wd)
    bd_p = jnp.zeros((1, _D), f32).at[0, :_NO].set(bd)
    Wad_p = jnp.zeros((_D, _D), f32).at[:, :_NA].set(Wad)
    bad_p = jnp.zeros((1, _D), f32).at[0, :_NA].set(bad)
    lc, pc = _head(h5, Wd_p, bd_p, _NO)
    la, pa = _head(h5, Wad_p, bad_p, _NA)

    cfg_logits = lc[:_N_CFG, :_NO]
    cfg_pred = pc[:_N_CFG, :_NO]
    ast_logits = la[_N_CFG:_N_CFG + _N_AST, :_NA]
    ast_pred = pa[_N_CFG:_N_CFG + _N_AST, :_NA]
    return (cfg_logits, cfg_pred, ast_logits, ast_pred)


# speedup vs baseline: 2.8430x; 2.4750x over previous
import functools

import jax
import jax.numpy as jnp
from jax import lax
from jax.experimental import pallas as pl
from jax.experimental.pallas import tpu as pltpu
from jax.experimental.pallas import tpu_sc as plsc

_N_CFG = 6000
_N_AST = 3800
_N_TEST = 200
_N = _N_CFG + _N_AST + _N_TEST
_E = 320000
_D = 128
_H = _D // 2
_VC = 1000
_VA = 1000
_NO = 10
_NA = 3

_NC = 2
_NS = 16
_NW = _NC * _NS

_N_ACC = 10240
_ECHUNK = 128
_NCHT = 2560
_FAST_CORE = 0
_CF = 120
_CS = 40
_CWIN = 40
_RPT = _N_ACC // _NS
_GPW = _N_ACC // _NW
_GCH = 80
_GN = _GPW // _GCH

@functools.cache
def _sc_kernels():
    mesh = plsc.VectorSubcoreMesh(core_axis_name="c", subcore_axis_name="s")

    nbuf = 2

    @functools.partial(
        pl.kernel,
        mesh=mesh,
        out_type=jax.ShapeDtypeStruct((_NC, _N_ACC, _D), jnp.float32),
        scratch_types=[
            pltpu.VMEM((_CWIN, _ECHUNK), jnp.int32),
            pltpu.VMEM((_CWIN, _ECHUNK), jnp.int32),
            pltpu.VMEM((nbuf, _ECHUNK, _D), jnp.float32),
            pltpu.VMEM((16, _D), jnp.float32),
            pltpu.VMEM_SHARED((_N_ACC, _D), jnp.float32),
        ] + [pltpu.SemaphoreType.DMA] * (2 * nbuf),
    )
    def segsum_k(p_hbm, src_hbm, dst_hbm, out_hbm, src_v, dst_v, gbuf, zbuf, acc, *sems):
        c = lax.axis_index("c")
        s = lax.axis_index("s")
        wid = s * _NC + c
        osems = sems[nbuf:]
        osem = osems[0]

        z16 = jnp.zeros((16,), jnp.float32)

        def _zf(i, carry):
            zbuf[i // 8, pl.ds((i % 8) * 16, 16)] = z16
            return carry

        lax.fori_loop(0, 16 * (_D // 16), _zf, 0)
        nz = _RPT // 16

        def _zc(k, carry):
            pltpu.async_copy(zbuf, acc.at[pl.ds(s * _RPT + k * 16, 16)], osem)
            return carry

        lax.fori_loop(0, nz, _zc, 0)

        def _zw(k, carry):
            pltpu.make_async_copy(zbuf, acc.at[pl.ds(s * _RPT, 16)], osem).wait()
            return carry

        lax.fori_loop(0, nz, _zw, 0)
        plsc.subcore_barrier()

        nst = jnp.where(c == _FAST_CORE, _CF // _CWIN, _CS // _CWIN)

        def _stage(h, carry):
            pltpu.sync_copy(src_hbm.at[wid, pl.ds(h * _CWIN, _CWIN)], src_v)
            pltpu.sync_copy(dst_hbm.at[wid, pl.ds(h * _CWIN, _CWIN)], dst_v)
            for b in range(nbuf):
                pltpu.async_copy(p_hbm.at[src_v.at[b]], gbuf.at[b], sems[b])

            def _step(jj, carry2):
                for b in range(nbuf):
                    j = jj * nbuf + b
                    pltpu.make_async_copy(p_hbm.at[src_v.at[j]], gbuf.at[b], sems[b]).wait()
                    pltpu.sync_copy(gbuf.at[b], acc.at[dst_v.at[j]], add=True)
                    pltpu.async_copy(p_hbm.at[src_v.at[j + nbuf]], gbuf.at[b], sems[b])
                return carry2

            lax.fori_loop(0, _CWIN // nbuf - 1, _step, 0)
            for b in range(nbuf):
                j = _CWIN - nbuf + b
                pltpu.make_async_copy(p_hbm.at[src_v.at[j]], gbuf.at[b], sems[b]).wait()
                pltpu.sync_copy(gbuf.at[b], acc.at[dst_v.at[j]], add=True)
            return carry

        lax.fori_loop(0, nst, _stage, 0)
        plsc.subcore_barrier()

        nout = _RPT // _ECHUNK
        for k in range(nout):
            b = k % nbuf
            r0 = s * _RPT + k * _ECHUNK
            if k >= nbuf:
                pltpu.make_async_copy(
                    gbuf.at[b], out_hbm.at[c, pl.ds(r0, _ECHUNK)], osems[b]).wait()
            pltpu.async_copy(acc.at[pl.ds(r0, _ECHUNK)], gbuf.at[b], sems[b]).wait()
            pltpu.async_copy(gbuf.at[b], out_hbm.at[c, pl.ds(r0, _ECHUNK)], osems[b])
        for k in range(nout - nbuf, nout):
            b = k % nbuf
            r0 = s * _RPT + k * _ECHUNK
            pltpu.make_async_copy(
                gbuf.at[b], out_hbm.at[c, pl.ds(r0, _ECHUNK)], osems[b]).wait()

    @functools.partial(
        pl.kernel,
        mesh=mesh,
        out_type=jax.ShapeDtypeStruct((_N_ACC, _D), jnp.float32),
        scratch_types=[
            pltpu.VMEM((_GN, _GCH), jnp.int32),
            pltpu.VMEM((_GCH, _D), jnp.float32),
            pltpu.SemaphoreType.DMA,
        ],
    )
    def emb_gather_k(tab_hbm, idx_hbm, out_hbm, idx_v, gbuf, sem):
        c = lax.axis_index("c")
        s = lax.axis_index("s")
        wid = s * _NC + c
        pltpu.sync_copy(idx_hbm.at[wid], idx_v)
        for k in range(_GN):
            pltpu.async_copy(tab_hbm.at[idx_v.at[k]], gbuf, sem).wait()
            pltpu.sync_copy(gbuf, out_hbm.at[pl.ds(wid * _GPW + k * _GCH, _GCH)])

    return segsum_k, emb_gather_k


def _segsum(p, src_r, dst_r):
    return _sc_kernels()[0](p, src_r, dst_r)


def _emb_gather(tab, gidx):
    return _sc_kernels()[1](tab, gidx)


_BR = 1024
_NBR = _N_ACC // _BR


def _fuse(a, b, bias, w, res=None):
    has_res = res is not None

    def body(*refs):
        if has_res:
            a_r, b_r, bias_r, w_r, res_r, h_r = refs
        else:
            a_r, b_r, bias_r, w_r, h_r = refs
        agg = a_r[...] + b_r[...]
        x = jnp.dot(agg, w_r[...], preferred_element_type=jnp.float32) + bias_r[...]
        x = jnp.maximum(x, 0.0)
        if has_res:
            x = x + res_r[...]
        h_r[...] = x

    row = pl.BlockSpec((_BR, _D), lambda i: (i, 0))
    one = pl.BlockSpec((1, _D), lambda i: (0, 0))
    ww = pl.BlockSpec((_D, _D), lambda i: (0, 0))
    in_specs = [row, row, one, ww] + ([row] if has_res else [])
    args = (a, b, bias, w) + ((res,) if has_res else ())
    return pl.pallas_call(
        body,
        grid=(_NBR,),
        in_specs=in_specs,
        out_specs=row,
        out_shape=jax.ShapeDtypeStruct((_N_ACC, _D), jnp.float32),
    )(*args)


def _add2(a, b):
    def body(a_r, b_r, o_r):
        o_r[...] = a_r[...] + b_r[...]

    row = pl.BlockSpec((_BR, _D), lambda i: (i, 0))
    return pl.pallas_call(
        body,
        grid=(_NBR,),
        in_specs=[row, row],
        out_specs=row,
        out_shape=jax.ShapeDtypeStruct((_N_ACC, _D), jnp.float32),
    )(a, b)


def _mm_bias(x, w, b, br):
    rows = x.shape[0]

    def body(x_r, w_r, b_r, o_r):
        o_r[...] = jnp.dot(x_r[...], w_r[...], preferred_element_type=jnp.float32) + b_r[...]

    return pl.pallas_call(
        body,
        grid=(rows // br,),
        in_specs=[
            pl.BlockSpec((br, _D), lambda i: (i, 0)),
            pl.BlockSpec((_D, _D), lambda i: (0, 0)),
            pl.BlockSpec((1, _D), lambda i: (0, 0)),
        ],
        out_specs=pl.BlockSpec((br, _D), lambda i: (i, 0)),
        out_shape=jax.ShapeDtypeStruct((rows, _D), jnp.float32),
    )(x, w, b)


def _head(x, w, b, nvalid):
    def body(x_r, w_r, b_r, l_r, p_r):
        l = jnp.dot(x_r[...], w_r[...], preferred_element_type=jnp.float32) + b_r[...]
        col = lax.broadcasted_iota(jnp.int32, l.shape, 1)
        mask = col < nvalid
        ml = jnp.where(mask, l, -1e30)
        mx = jnp.max(ml, axis=1, keepdims=True)
        e = jnp.where(mask, jnp.exp(ml - mx), 0.0)
        ssum = jnp.sum(e, axis=1, keepdims=True)
        l_r[...] = l
        p_r[...] = e / ssum

    row = pl.BlockSpec((_BR, _D), lambda i: (i, 0))
    return pl.pallas_call(
        body,
        grid=(_NBR,),
        in_specs=[row, pl.BlockSpec((_D, _D), lambda i: (0, 0)),
                  pl.BlockSpec((1, _D), lambda i: (0, 0))],
        out_specs=[row, row],
        out_shape=[jax.ShapeDtypeStruct((_N_ACC, _D), jnp.float32)] * 2,
    )(x, w, b)


def _layout_edges(x, pad_arr, fillval):
    nreal = _E // _ECHUNK
    npadc = _NCHT - nreal
    ch = x.reshape(nreal, _ECHUNK)
    padc = pad_arr.reshape(npadc, _ECHUNK)
    nreal_fast = _NS * _CF - npadc
    fast = jnp.concatenate([ch[:nreal_fast], padc]).reshape(_NS, _CF, _ECHUNK)
    slow = ch[nreal_fast:].reshape(_NS, _CS, _ECHUNK)
    slow = jnp.concatenate(
        [slow, jnp.full((_NS, _CF - _CS, _ECHUNK), fillval, jnp.int32)], axis=1)
    parts = [None, None]
    parts[_FAST_CORE] = fast
    parts[1 - _FAST_CORE] = slow
    return jnp.stack(parts, axis=1).reshape(_NW, _CF, _ECHUNK)


def kernel(cfg_label, cfg_content, ast_label, ast_content, edge_index,
           c_lbl_emb, Wc, bc, a_lbl_emb, Wa, ba, t_emb,
           W1, b1, W2, b2, W3, b3, W4, b4, W5, b5,
           Wd, bd, Wad, bad):
    f32 = jnp.float32

    src = edge_index[0].astype(jnp.int32)
    dst = edge_index[1].astype(jnp.int32)
    npad = _NCHT * _ECHUNK - _E
    pad_src = jnp.arange(npad, dtype=jnp.int32) % _N
    pad_dst = _N + (jnp.arange(npad, dtype=jnp.int32) % (_N_ACC - _N))
    src_r = _layout_edges(src, pad_src, 0)
    dst_r = _layout_edges(dst, pad_dst, _N)

    tab = jnp.zeros((2048, _D), f32)
    tab = tab.at[:_VC, :_H].set(c_lbl_emb)
    tab = tab.at[_VC:_VC + _VA, :_H].set(a_lbl_emb)
    tab = tab.at[_VC + _VA, :].set(t_emb)
    gidx = jnp.concatenate([
        cfg_label.astype(jnp.int32),
        ast_label.astype(jnp.int32) + _VC,
        jnp.full((_N_TEST,), _VC + _VA, jnp.int32),
        jnp.full((_N_ACC - _N,), _VC + _VA + 1, jnp.int32),
    ]).reshape(_NW, _GN, _GCH)

    Wc_p = jnp.zeros((_D, _D), f32).at[:, _H:].set(Wc)
    bc_p = jnp.zeros((1, _D), f32).at[0, _H:].set(bc)
    Wa_p = jnp.zeros((_D, _D), f32).at[:, _H:].set(Wa)
    ba_p = jnp.zeros((1, _D), f32).at[0, _H:].set(ba)
    cfgc = jnp.concatenate([cfg_content, jnp.zeros((6144 - _N_CFG, _D), f32)])
    astc = jnp.concatenate([ast_content, jnp.zeros((3840 - _N_AST, _D), f32)])

    g = _emb_gather(tab, gidx)
    cp_cfg = _mm_bias(cfgc, Wc_p, bc_p, 768)
    cp_ast = _mm_bias(astc, Wa_p, ba_p, 768)
    cp = jnp.concatenate([cp_cfg[:_N_CFG], cp_ast[:_N_AST],
                          jnp.zeros((_N_ACC - _N, _D), f32)])

    h0 = _add2(g, cp)
    a1 = _segsum(h0, src_r, dst_r)
    h1 = _fuse(a1[0], a1[1], b1.reshape(1, -1), W1)
    a2 = _segsum(h1, src_r, dst_r)
    h2 = _fuse(a2[0], a2[1], b2.reshape(1, -1), W2, res=h1)
    a3 = _segsum(h2, src_r, dst_r)
    h3 = _fuse(a3[0], a3[1], b3.reshape(1, -1), W3)
    a4 = _segsum(h3, src_r, dst_r)
    h4 = _fuse(a4[0], a4[1], b4.reshape(1, -1), W4, res=h3)
    a5 = _segsum(h4, src_r, dst_r)
    h5 = _fuse(a5[0], a5[1], b5.reshape(1, -1), W5)

    Wd_p = jnp.zeros((_D, _D), f32).at[:, :_NO].set(Wd)
    bd_p = jnp.zeros((1, _D), f32).at[0, :_NO].set(bd)
    Wad_p = jnp.zeros((_D, _D), f32).at[:, :_NA].set(Wad)
    bad_p = jnp.zeros((1, _D), f32).at[0, :_NA].set(bad)
    lc, pc = _head(h5, Wd_p, bd_p, _NO)
    la, pa = _head(h5, Wad_p, bad_p, _NA)

    cfg_logits = lc[:_N_CFG, :_NO]
    cfg_pred = pc[:_N_CFG, :_NO]
    ast_logits = la[_N_CFG:_N_CFG + _N_AST, :_NA]
    ast_pred = pa[_N_CFG:_N_CFG + _N_AST, :_NA]
    return (cfg_logits, cfg_pred, ast_logits, ast_pred)


# speedup vs baseline: 3.9157x; 1.3773x over previous
import functools

import jax
import jax.numpy as jnp
from jax import lax
from jax.experimental import pallas as pl
from jax.experimental.pallas import tpu as pltpu
from jax.experimental.pallas import tpu_sc as plsc

_N_CFG = 6000
_N_AST = 3800
_N_TEST = 200
_N = _N_CFG + _N_AST + _N_TEST
_E = 320000
_D = 128
_H = _D // 2
_VC = 1000
_VA = 1000
_NO = 10
_NA = 3

_NC = 2
_NS = 16
_NW = _NC * _NS

_N_ACC = 10240
_ECHUNK = 128
_NCHT = 2560
_FAST_CORE = 0
_CF = 80
_CS = 80
_CWIN = 40
_RPT = _N_ACC // _NS
_GPW = _N_ACC // _NW
_GCH = 80
_GN = _GPW // _GCH

@functools.cache
def _sc_kernels():
    mesh = plsc.VectorSubcoreMesh(core_axis_name="c", subcore_axis_name="s")

    nbuf = 2

    @functools.partial(
        pl.kernel,
        mesh=mesh,
        out_type=jax.ShapeDtypeStruct((_NC, _N_ACC, _D), jnp.float32),
        scratch_types=[
            pltpu.VMEM((_CWIN, _ECHUNK), jnp.int32),
            pltpu.VMEM((_CWIN, _ECHUNK), jnp.int32),
            pltpu.VMEM((nbuf, _ECHUNK, _D), jnp.float32),
            pltpu.VMEM((16, _D), jnp.float32),
            pltpu.VMEM_SHARED((_N_ACC, _D), jnp.float32),
        ] + [pltpu.SemaphoreType.DMA] * (2 * nbuf),
    )
    def segsum_k(p_hbm, src_hbm, dst_hbm, out_hbm, src_v, dst_v, gbuf, zbuf, acc, *sems):
        c = lax.axis_index("c")
        s = lax.axis_index("s")
        wid = s * _NC + c
        osems = sems[nbuf:]
        osem = osems[0]

        z16 = jnp.zeros((16,), jnp.float32)

        def _zf(i, carry):
            zbuf[i // 8, pl.ds((i % 8) * 16, 16)] = z16
            return carry

        lax.fori_loop(0, 16 * (_D // 16), _zf, 0)
        nz = _RPT // 16

        def _zc(k, carry):
            pltpu.async_copy(zbuf, acc.at[pl.ds(s * _RPT + k * 16, 16)], osem)
            return carry

        lax.fori_loop(0, nz, _zc, 0)

        def _zw(k, carry):
            pltpu.make_async_copy(zbuf, acc.at[pl.ds(s * _RPT, 16)], osem).wait()
            return carry

        lax.fori_loop(0, nz, _zw, 0)
        plsc.subcore_barrier()

        nst = jnp.where(c == _FAST_CORE, _CF // _CWIN, _CS // _CWIN)

        def _stage(h, carry):
            pltpu.sync_copy(src_hbm.at[wid, pl.ds(h * _CWIN, _CWIN)], src_v)
            pltpu.sync_copy(dst_hbm.at[wid, pl.ds(h * _CWIN, _CWIN)], dst_v)
            for b in range(nbuf):
                pltpu.async_copy(p_hbm.at[src_v.at[b]], gbuf.at[b], sems[b])

            def _step(jj, carry2):
                for b in range(nbuf):
                    j = jj * nbuf + b
                    pltpu.make_async_copy(p_hbm.at[src_v.at[j]], gbuf.at[b], sems[b]).wait()
                    pltpu.sync_copy(gbuf.at[b], acc.at[dst_v.at[j]], add=True)
                    pltpu.async_copy(p_hbm.at[src_v.at[j + nbuf]], gbuf.at[b], sems[b])
                return carry2

            lax.fori_loop(0, _CWIN // nbuf - 1, _step, 0)
            for b in range(nbuf):
                j = _CWIN - nbuf + b
                pltpu.make_async_copy(p_hbm.at[src_v.at[j]], gbuf.at[b], sems[b]).wait()
                pltpu.sync_copy(gbuf.at[b], acc.at[dst_v.at[j]], add=True)
            return carry

        lax.fori_loop(0, nst, _stage, 0)
        plsc.subcore_barrier()

        nout = _RPT // _ECHUNK
        for k in range(nout):
            b = k % nbuf
            r0 = s * _RPT + k * _ECHUNK
            if k >= nbuf:
                pltpu.make_async_copy(
                    gbuf.at[b], out_hbm.at[c, pl.ds(r0, _ECHUNK)], osems[b]).wait()
            pltpu.async_copy(acc.at[pl.ds(r0, _ECHUNK)], gbuf.at[b], sems[b]).wait()
            pltpu.async_copy(gbuf.at[b], out_hbm.at[c, pl.ds(r0, _ECHUNK)], osems[b])
        for k in range(nout - nbuf, nout):
            b = k % nbuf
            r0 = s * _RPT + k * _ECHUNK
            pltpu.make_async_copy(
                gbuf.at[b], out_hbm.at[c, pl.ds(r0, _ECHUNK)], osems[b]).wait()

    @functools.partial(
        pl.kernel,
        mesh=mesh,
        out_type=jax.ShapeDtypeStruct((_N_ACC, _D), jnp.float32),
        scratch_types=[
            pltpu.VMEM((_GN, _GCH), jnp.int32),
            pltpu.VMEM((_GCH, _D), jnp.float32),
            pltpu.SemaphoreType.DMA,
        ],
    )
    def emb_gather_k(tab_hbm, idx_hbm, out_hbm, idx_v, gbuf, sem):
        c = lax.axis_index("c")
        s = lax.axis_index("s")
        wid = s * _NC + c
        pltpu.sync_copy(idx_hbm.at[wid], idx_v)
        for k in range(_GN):
            pltpu.async_copy(tab_hbm.at[idx_v.at[k]], gbuf, sem).wait()
            pltpu.sync_copy(gbuf, out_hbm.at[pl.ds(wid * _GPW + k * _GCH, _GCH)])

    return segsum_k, emb_gather_k


def _segsum(p, src_r, dst_r):
    return _sc_kernels()[0](p, src_r, dst_r)


def _emb_gather(tab, gidx):
    return _sc_kernels()[1](tab, gidx)


_BR = 1024
_NBR = _N_ACC // _BR


def _fuse(a2, bias, w, res=None):
    has_res = res is not None

    def body(*refs):
        if has_res:
            a0_r, a1_r, bias_r, w_r, res_r, h_r = refs
        else:
            a0_r, a1_r, bias_r, w_r, h_r = refs
        agg = a0_r[0] + a1_r[0]
        x = jnp.dot(agg, w_r[...], preferred_element_type=jnp.float32) + bias_r[...]
        x = jnp.maximum(x, 0.0)
        if has_res:
            x = x + res_r[...]
        h_r[...] = x

    row = pl.BlockSpec((_BR, _D), lambda i: (i, 0))
    pl0 = pl.BlockSpec((1, _BR, _D), lambda i: (0, i, 0))
    pl1 = pl.BlockSpec((1, _BR, _D), lambda i: (1, i, 0))
    one = pl.BlockSpec((1, _D), lambda i: (0, 0))
    ww = pl.BlockSpec((_D, _D), lambda i: (0, 0))
    in_specs = [pl0, pl1, one, ww] + ([row] if has_res else [])
    args = (a2, a2, bias, w) + ((res,) if has_res else ())
    return pl.pallas_call(
        body,
        grid=(_NBR,),
        in_specs=in_specs,
        out_specs=row,
        out_shape=jax.ShapeDtypeStruct((_N_ACC, _D), jnp.float32),
    )(*args)


def _add2(a, b):
    def body(a_r, b_r, o_r):
        o_r[...] = a_r[...] + b_r[...]

    row = pl.BlockSpec((_BR, _D), lambda i: (i, 0))
    return pl.pallas_call(
        body,
        grid=(_NBR,),
        in_specs=[row, row],
        out_specs=row,
        out_shape=jax.ShapeDtypeStruct((_N_ACC, _D), jnp.float32),
    )(a, b)


def _mm_bias(x, w, b, br):
    rows = x.shape[0]

    def body(x_r, w_r, b_r, o_r):
        o_r[...] = jnp.dot(x_r[...], w_r[...], preferred_element_type=jnp.float32) + b_r[...]

    return pl.pallas_call(
        body,
        grid=(rows // br,),
        in_specs=[
            pl.BlockSpec((br, _D), lambda i: (i, 0)),
            pl.BlockSpec((_D, _D), lambda i: (0, 0)),
            pl.BlockSpec((1, _D), lambda i: (0, 0)),
        ],
        out_specs=pl.BlockSpec((br, _D), lambda i: (i, 0)),
        out_shape=jax.ShapeDtypeStruct((rows, _D), jnp.float32),
    )(x, w, b)


def _head(x, w, b, nvalid):
    def body(x_r, w_r, b_r, l_r, p_r):
        l = jnp.dot(x_r[...], w_r[...], preferred_element_type=jnp.float32) + b_r[...]
        col = lax.broadcasted_iota(jnp.int32, l.shape, 1)
        mask = col < nvalid
        ml = jnp.where(mask, l, -1e30)
        mx = jnp.max(ml, axis=1, keepdims=True)
        e = jnp.where(mask, jnp.exp(ml - mx), 0.0)
        ssum = jnp.sum(e, axis=1, keepdims=True)
        l_r[...] = l
        p_r[...] = e / ssum

    row = pl.BlockSpec((_BR, _D), lambda i: (i, 0))
    return pl.pallas_call(
        body,
        grid=(_NBR,),
        in_specs=[row, pl.BlockSpec((_D, _D), lambda i: (0, 0)),
                  pl.BlockSpec((1, _D), lambda i: (0, 0))],
        out_specs=[row, row],
        out_shape=[jax.ShapeDtypeStruct((_N_ACC, _D), jnp.float32)] * 2,
    )(x, w, b)


def _layout_edges(x, pad_arr, fillval):
    nreal = _E // _ECHUNK
    npadc = _NCHT - nreal
    ch = x.reshape(nreal, _ECHUNK)
    padc = pad_arr.reshape(npadc, _ECHUNK)
    nreal_fast = _NS * _CF - npadc
    fast = jnp.concatenate([ch[:nreal_fast], padc]).reshape(_NS, _CF, _ECHUNK)
    slow = ch[nreal_fast:].reshape(_NS, _CS, _ECHUNK)
    slow = jnp.concatenate(
        [slow, jnp.full((_NS, _CF - _CS, _ECHUNK), fillval, jnp.int32)], axis=1)
    parts = [None, None]
    parts[_FAST_CORE] = fast
    parts[1 - _FAST_CORE] = slow
    return jnp.stack(parts, axis=1).reshape(_NW, _CF, _ECHUNK)


def kernel(cfg_label, cfg_content, ast_label, ast_content, edge_index,
           c_lbl_emb, Wc, bc, a_lbl_emb, Wa, ba, t_emb,
           W1, b1, W2, b2, W3, b3, W4, b4, W5, b5,
           Wd, bd, Wad, bad):
    f32 = jnp.float32

    src = edge_index[0].astype(jnp.int32)
    dst = edge_index[1].astype(jnp.int32)
    npad = _NCHT * _ECHUNK - _E
    pad_src = jnp.arange(npad, dtype=jnp.int32) % _N
    pad_dst = _N + (jnp.arange(npad, dtype=jnp.int32) % (_N_ACC - _N))
    src_r = _layout_edges(src, pad_src, 0)
    dst_r = _layout_edges(dst, pad_dst, _N)

    tab = jnp.zeros((2048, _D), f32)
    tab = tab.at[:_VC, :_H].set(c_lbl_emb)
    tab = tab.at[_VC:_VC + _VA, :_H].set(a_lbl_emb)
    tab = tab.at[_VC + _VA, :].set(t_emb)
    gidx = jnp.concatenate([
        cfg_label.astype(jnp.int32),
        ast_label.astype(jnp.int32) + _VC,
        jnp.full((_N_TEST,), _VC + _VA, jnp.int32),
        jnp.full((_N_ACC - _N,), _VC + _VA + 1, jnp.int32),
    ]).reshape(_NW, _GN, _GCH)

    Wc_p = jnp.zeros((_D, _D), f32).at[:, _H:].set(Wc)
    bc_p = jnp.zeros((1, _D), f32).at[0, _H:].set(bc)
    Wa_p = jnp.zeros((_D, _D), f32).at[:, _H:].set(Wa)
    ba_p = jnp.zeros((1, _D), f32).at[0, _H:].set(ba)
    cfgc = jnp.concatenate([cfg_content, jnp.zeros((6144 - _N_CFG, _D), f32)])
    astc = jnp.concatenate([ast_content, jnp.zeros((3840 - _N_AST, _D), f32)])

    g = _emb_gather(tab, gidx)
    cp_cfg = _mm_bias(cfgc, Wc_p, bc_p, 768)
    cp_ast = _mm_bias(astc, Wa_p, ba_p, 768)
    cp = jnp.concatenate([cp_cfg[:_N_CFG], cp_ast[:_N_AST],
                          jnp.zeros((_N_ACC - _N, _D), f32)])

    h0 = _add2(g, cp)
    a1 = _segsum(h0, src_r, dst_r)
    h1 = _fuse(a1, b1.reshape(1, -1), W1)
    a2 = _segsum(h1, src_r, dst_r)
    h2 = _fuse(a2, b2.reshape(1, -1), W2, res=h1)
    a3 = _segsum(h2, src_r, dst_r)
    h3 = _fuse(a3, b3.reshape(1, -1), W3)
    a4 = _segsum(h3, src_r, dst_r)
    h4 = _fuse(a4, b4.reshape(1, -1), W4, res=h3)
    a5 = _segsum(h4, src_r, dst_r)
    h5 = _fuse(a5, b5.reshape(1, -1), W5)

    Wd_p = jnp.zeros((_D, _D), f32).at[:, :_NO].set(Wd)
    bd_p = jnp.zeros((1, _D), f32).at[0, :_NO].set(bd)
    Wad_p = jnp.zeros((_D, _D), f32).at[:, :_NA].set(Wad)
    bad_p = jnp.zeros((1, _D), f32).at[0, :_NA].set(bad)
    lc, pc = _head(h5, Wd_p, bd_p, _NO)
    la, pa = _head(h5, Wad_p, bad_p, _NA)

    cfg_logits = lc[:_N_CFG, :_NO]
    cfg_pred = pc[:_N_CFG, :_NO]
    ast_logits = la[_N_CFG:_N_CFG + _N_AST, :_NA]
    ast_pred = pa[_N_CFG:_N_CFG + _N_AST, :_NA]
    return (cfg_logits, cfg_pred, ast_logits, ast_pred)


# speedup vs baseline: 4.0148x; 1.0253x over previous
import functools

import jax
import jax.numpy as jnp
from jax import lax
from jax.experimental import pallas as pl
from jax.experimental.pallas import tpu as pltpu
from jax.experimental.pallas import tpu_sc as plsc

_N_CFG = 6000
_N_AST = 3800
_N_TEST = 200
_N = _N_CFG + _N_AST + _N_TEST
_E = 320000
_D = 128
_H = _D // 2
_VC = 1000
_VA = 1000
_NO = 10
_NA = 3

_NC = 2
_NS = 16
_NW = _NC * _NS

_N_ACC = 10240
_ECHUNK = 128
_NCHT = 2560
_FAST_CORE = 0
_CF = 80
_CS = 80
_CWIN = 40
_RPT = _N_ACC // _NS
_GPW = _N_ACC // _NW
_GCH = 80
_GN = _GPW // _GCH

@functools.cache
def _sc_kernels():
    mesh = plsc.VectorSubcoreMesh(core_axis_name="c", subcore_axis_name="s")

    nbuf = 2

    @functools.partial(
        pl.kernel,
        mesh=mesh,
        out_type=jax.ShapeDtypeStruct((_NC, _N_ACC, _D), jnp.float32),
        scratch_types=[
            pltpu.VMEM((_CWIN, _ECHUNK), jnp.int32),
            pltpu.VMEM((_CWIN, _ECHUNK), jnp.int32),
            pltpu.VMEM((nbuf, _ECHUNK, _D), jnp.float32),
            pltpu.VMEM((16, _D), jnp.float32),
            pltpu.VMEM_SHARED((_N_ACC, _D), jnp.float32),
        ] + [pltpu.SemaphoreType.DMA] * (2 * nbuf),
    )
    def segsum_k(p_hbm, src_hbm, dst_hbm, out_hbm, src_v, dst_v, gbuf, zbuf, acc, *sems):
        c = lax.axis_index("c")
        s = lax.axis_index("s")
        wid = s * _NC + c
        osems = sems[nbuf:]
        osem = osems[0]

        z16 = jnp.zeros((16,), jnp.float32)

        def _zf(i, carry):
            zbuf[i // 8, pl.ds((i % 8) * 16, 16)] = z16
            return carry

        lax.fori_loop(0, 16 * (_D // 16), _zf, 0)
        nz = _RPT // 16

        def _zc(k, carry):
            pltpu.async_copy(zbuf, acc.at[pl.ds(s * _RPT + k * 16, 16)], osem)
            return carry

        lax.fori_loop(0, nz, _zc, 0)

        def _zw(k, carry):
            pltpu.make_async_copy(zbuf, acc.at[pl.ds(s * _RPT, 16)], osem).wait()
            return carry

        lax.fori_loop(0, nz, _zw, 0)
        plsc.subcore_barrier()

        nst = jnp.where(c == _FAST_CORE, _CF // _CWIN, _CS // _CWIN)

        def _stage(h, carry):
            pltpu.sync_copy(src_hbm.at[wid, pl.ds(h * _CWIN, _CWIN)], src_v)
            pltpu.sync_copy(dst_hbm.at[wid, pl.ds(h * _CWIN, _CWIN)], dst_v)
            for b in range(nbuf):
                pltpu.async_copy(p_hbm.at[src_v.at[b]], gbuf.at[b], sems[b])

            def _step(jj, carry2):
                for b in range(nbuf):
                    j = jj * nbuf + b
                    pltpu.make_async_copy(p_hbm.at[src_v.at[j]], gbuf.at[b], sems[b]).wait()
                    pltpu.sync_copy(gbuf.at[b], acc.at[dst_v.at[j]], add=True)
                    pltpu.async_copy(p_hbm.at[src_v.at[j + nbuf]], gbuf.at[b], sems[b])
                return carry2

            lax.fori_loop(0, _CWIN // nbuf - 1, _step, 0)
            for b in range(nbuf):
                j = _CWIN - nbuf + b
                pltpu.make_async_copy(p_hbm.at[src_v.at[j]], gbuf.at[b], sems[b]).wait()
                pltpu.sync_copy(gbuf.at[b], acc.at[dst_v.at[j]], add=True)
            return carry

        lax.fori_loop(0, nst, _stage, 0)
        plsc.subcore_barrier()

        nout = _RPT // _ECHUNK
        for k in range(nout):
            b = k % nbuf
            r0 = s * _RPT + k * _ECHUNK
            if k >= nbuf:
                pltpu.make_async_copy(
                    gbuf.at[b], out_hbm.at[c, pl.ds(r0, _ECHUNK)], osems[b]).wait()
            pltpu.async_copy(acc.at[pl.ds(r0, _ECHUNK)], gbuf.at[b], sems[b]).wait()
            pltpu.async_copy(gbuf.at[b], out_hbm.at[c, pl.ds(r0, _ECHUNK)], osems[b])
        for k in range(nout - nbuf, nout):
            b = k % nbuf
            r0 = s * _RPT + k * _ECHUNK
            pltpu.make_async_copy(
                gbuf.at[b], out_hbm.at[c, pl.ds(r0, _ECHUNK)], osems[b]).wait()

    @functools.partial(
        pl.kernel,
        mesh=mesh,
        out_type=jax.ShapeDtypeStruct((_N_ACC, _D), jnp.float32),
        scratch_types=[
            pltpu.VMEM((_GN, _GCH), jnp.int32),
            pltpu.VMEM((_GCH, _D), jnp.float32),
            pltpu.SemaphoreType.DMA,
        ],
    )
    def emb_gather_k(tab_hbm, idx_hbm, out_hbm, idx_v, gbuf, sem):
        c = lax.axis_index("c")
        s = lax.axis_index("s")
        wid = s * _NC + c
        pltpu.sync_copy(idx_hbm.at[wid], idx_v)
        for k in range(_GN):
            pltpu.async_copy(tab_hbm.at[idx_v.at[k]], gbuf, sem).wait()
            pltpu.sync_copy(gbuf, out_hbm.at[pl.ds(wid * _GPW + k * _GCH, _GCH)])

    return segsum_k, emb_gather_k


def _segsum(p, src_r, dst_r):
    return _sc_kernels()[0](p, src_r, dst_r)


def _emb_gather(tab, gidx):
    return _sc_kernels()[1](tab, gidx)


_BR = 1024
_NBR = _N_ACC // _BR


def _fuse(a2, bias, w, res=None):
    has_res = res is not None

    def body(*refs):
        if has_res:
            a0_r, a1_r, bias_r, w_r, res_r, h_r = refs
        else:
            a0_r, a1_r, bias_r, w_r, h_r = refs
        agg = a0_r[0] + a1_r[0]
        x = jnp.dot(agg, w_r[...], preferred_element_type=jnp.float32) + bias_r[...]
        x = jnp.maximum(x, 0.0)
        if has_res:
            x = x + res_r[...]
        h_r[...] = x

    row = pl.BlockSpec((_BR, _D), lambda i: (i, 0))
    pl0 = pl.BlockSpec((1, _BR, _D), lambda i: (0, i, 0))
    pl1 = pl.BlockSpec((1, _BR, _D), lambda i: (1, i, 0))
    one = pl.BlockSpec((1, _D), lambda i: (0, 0))
    ww = pl.BlockSpec((_D, _D), lambda i: (0, 0))
    in_specs = [pl0, pl1, one, ww] + ([row] if has_res else [])
    args = (a2, a2, bias, w) + ((res,) if has_res else ())
    return pl.pallas_call(
        body,
        grid=(_NBR,),
        in_specs=in_specs,
        out_specs=row,
        out_shape=jax.ShapeDtypeStruct((_N_ACC, _D), jnp.float32),
    )(*args)


def _add2(a, b):
    def body(a_r, b_r, o_r):
        o_r[...] = a_r[...] + b_r[...]

    row = pl.BlockSpec((_BR, _D), lambda i: (i, 0))
    return pl.pallas_call(
        body,
        grid=(_NBR,),
        in_specs=[row, row],
        out_specs=row,
        out_shape=jax.ShapeDtypeStruct((_N_ACC, _D), jnp.float32),
    )(a, b)


def _mm_bias(x, w, b, br):
    rows = x.shape[0]

    def body(x_r, w_r, b_r, o_r):
        o_r[...] = jnp.dot(x_r[...], w_r[...], preferred_element_type=jnp.float32) + b_r[...]

    return pl.pallas_call(
        body,
        grid=(rows // br,),
        in_specs=[
            pl.BlockSpec((br, _D), lambda i: (i, 0)),
            pl.BlockSpec((_D, _D), lambda i: (0, 0)),
            pl.BlockSpec((1, _D), lambda i: (0, 0)),
        ],
        out_specs=pl.BlockSpec((br, _D), lambda i: (i, 0)),
        out_shape=jax.ShapeDtypeStruct((rows, _D), jnp.float32),
    )(x, w, b)


def _final(a2, bias, w, wc, bc_, wa, ba_):
    def head(h, w_r, b_r, nvalid):
        l = jnp.dot(h, w_r[...], preferred_element_type=jnp.float32) + b_r[...]
        col = lax.broadcasted_iota(jnp.int32, l.shape, 1)
        mask = col < nvalid
        ml = jnp.where(mask, l, -1e30)
        mx = jnp.max(ml, axis=1, keepdims=True)
        e = jnp.where(mask, jnp.exp(ml - mx), 0.0)
        return l, e / jnp.sum(e, axis=1, keepdims=True)

    def body(a0_r, a1_r, bias_r, w_r, wc_r, bc_r, wa_r, ba_r,
             lc_r, pc_r, la_r, pa_r):
        agg = a0_r[0] + a1_r[0]
        h = jnp.maximum(
            jnp.dot(agg, w_r[...], preferred_element_type=jnp.float32) + bias_r[...],
            0.0)
        lc_r[...], pc_r[...] = head(h, wc_r, bc_r, _NO)
        la_r[...], pa_r[...] = head(h, wa_r, ba_r, _NA)

    row = pl.BlockSpec((_BR, _D), lambda i: (i, 0))
    pl0 = pl.BlockSpec((1, _BR, _D), lambda i: (0, i, 0))
    pl1 = pl.BlockSpec((1, _BR, _D), lambda i: (1, i, 0))
    one = pl.BlockSpec((1, _D), lambda i: (0, 0))
    ww = pl.BlockSpec((_D, _D), lambda i: (0, 0))
    return pl.pallas_call(
        body,
        grid=(_NBR,),
        in_specs=[pl0, pl1, one, ww, ww, one, ww, one],
        out_specs=[row, row, row, row],
        out_shape=[jax.ShapeDtypeStruct((_N_ACC, _D), jnp.float32)] * 4,
    )(a2, a2, bias, w, wc, bc_, wa, ba_)


def _layout_edges(x, pad_arr, fillval):
    nreal = _E // _ECHUNK
    npadc = _NCHT - nreal
    ch = x.reshape(nreal, _ECHUNK)
    padc = pad_arr.reshape(npadc, _ECHUNK)
    nreal_fast = _NS * _CF - npadc
    fast = jnp.concatenate([ch[:nreal_fast], padc]).reshape(_NS, _CF, _ECHUNK)
    slow = ch[nreal_fast:].reshape(_NS, _CS, _ECHUNK)
    slow = jnp.concatenate(
        [slow, jnp.full((_NS, _CF - _CS, _ECHUNK), fillval, jnp.int32)], axis=1)
    parts = [None, None]
    parts[_FAST_CORE] = fast
    parts[1 - _FAST_CORE] = slow
    return jnp.stack(parts, axis=1).reshape(_NW, _CF, _ECHUNK)


def kernel(cfg_label, cfg_content, ast_label, ast_content, edge_index,
           c_lbl_emb, Wc, bc, a_lbl_emb, Wa, ba, t_emb,
           W1, b1, W2, b2, W3, b3, W4, b4, W5, b5,
           Wd, bd, Wad, bad):
    f32 = jnp.float32

    src = edge_index[0].astype(jnp.int32)
    dst = edge_index[1].astype(jnp.int32)
    npad = _NCHT * _ECHUNK - _E
    pad_src = jnp.arange(npad, dtype=jnp.int32) % _N
    pad_dst = _N + (jnp.arange(npad, dtype=jnp.int32) % (_N_ACC - _N))
    src_r = _layout_edges(src, pad_src, 0)
    dst_r = _layout_edges(dst, pad_dst, _N)

    tab = jnp.zeros((2048, _D), f32)
    tab = tab.at[:_VC, :_H].set(c_lbl_emb)
    tab = tab.at[_VC:_VC + _VA, :_H].set(a_lbl_emb)
    tab = tab.at[_VC + _VA, :].set(t_emb)
    gidx = jnp.concatenate([
        cfg_label.astype(jnp.int32),
        ast_label.astype(jnp.int32) + _VC,
        jnp.full((_N_TEST,), _VC + _VA, jnp.int32),
        jnp.full((_N_ACC - _N,), _VC + _VA + 1, jnp.int32),
    ]).reshape(_NW, _GN, _GCH)

    Wc_p = jnp.zeros((_D, _D), f32).at[:, _H:].set(Wc)
    bc_p = jnp.zeros((1, _D), f32).at[0, _H:].set(bc)
    Wa_p = jnp.zeros((_D, _D), f32).at[:, _H:].set(Wa)
    ba_p = jnp.zeros((1, _D), f32).at[0, _H:].set(ba)
    cfgc = jnp.concatenate([cfg_content, jnp.zeros((6144 - _N_CFG, _D), f32)])
    astc = jnp.concatenate([ast_content, jnp.zeros((3840 - _N_AST, _D), f32)])

    g = _emb_gather(tab, gidx)
    cp_cfg = _mm_bias(cfgc, Wc_p, bc_p, 768)
    cp_ast = _mm_bias(astc, Wa_p, ba_p, 768)
    cp = jnp.concatenate([cp_cfg[:_N_CFG], cp_ast[:_N_AST],
                          jnp.zeros((_N_ACC - _N, _D), f32)])

    h0 = _add2(g, cp)
    a1 = _segsum(h0, src_r, dst_r)
    h1 = _fuse(a1, b1.reshape(1, -1), W1)
    a2 = _segsum(h1, src_r, dst_r)
    h2 = _fuse(a2, b2.reshape(1, -1), W2, res=h1)
    a3 = _segsum(h2, src_r, dst_r)
    h3 = _fuse(a3, b3.reshape(1, -1), W3)
    a4 = _segsum(h3, src_r, dst_r)
    h4 = _fuse(a4, b4.reshape(1, -1), W4, res=h3)
    a5 = _segsum(h4, src_r, dst_r)

    Wd_p = jnp.zeros((_D, _D), f32).at[:, :_NO].set(Wd)
    bd_p = jnp.zeros((1, _D), f32).at[0, :_NO].set(bd)
    Wad_p = jnp.zeros((_D, _D), f32).at[:, :_NA].set(Wad)
    bad_p = jnp.zeros((1, _D), f32).at[0, :_NA].set(bad)
    lc, pc, la, pa = _final(a5, b5.reshape(1, -1), W5, Wd_p, bd_p, Wad_p, bad_p)

    cfg_logits = lc[:_N_CFG, :_NO]
    cfg_pred = pc[:_N_CFG, :_NO]
    ast_logits = la[_N_CFG:_N_CFG + _N_AST, :_NA]
    ast_pred = pa[_N_CFG:_N_CFG + _N_AST, :_NA]
    return (cfg_logits, cfg_pred, ast_logits, ast_pred)
